# Initial kernel scaffold; baseline (speedup 1.0000x reference)
import jax
import jax.numpy as jnp
from jax.experimental import pallas as pl


def kernel(q_pts, s_pts, neighb_inds, x, weights, kernel_points):
    raise NotImplementedError("write your pallas kernel here")


# speedup vs baseline: 1.6205x; 1.6205x over previous
import functools
import math

import jax
import jax.numpy as jnp
from jax import lax
from jax.experimental import pallas as pl
from jax.experimental.pallas import tpu as pltpu
from jax.experimental.pallas import tpu_sc as plsc

N = 10000
M = 10000
H = 32
IN = 128
OUT = 128
K = 27
KS = 3
P = 3
RADIUS = 0.5
KP_EXTENT = 2.0 * RADIUS / (KS - 1) / math.sqrt(P)

BQ = 256
NP_ = 10240
NBLK = NP_ // BQ
B = NP_ * H

NW = 32
NC = 2
BPW = B // NW
CHUNK = 256
ROUNDS = BPW // CHUNK
STREAMS = CHUNK // 128
IDX_ROWS = BPW // 128


def _sc_gather_kernel(x_hbm, sp_hbm, idx_hbm, outx_hbm, outs_hbm,
                      idx_v, xbuf, sbuf, semx, sems):
    wid = lax.axis_index("s") * NC + lax.axis_index("c")
    base = wid * BPW
    pltpu.sync_copy(idx_hbm.at[wid], idx_v)

    def round_body(r, _):
        copies = []
        for j in range(STREAMS):
            row = r * STREAMS + j
            copies.append(pltpu.async_copy(
                x_hbm.at[idx_v.at[row]],
                xbuf.at[pl.ds(j * 128, 128)], semx))
            copies.append(pltpu.async_copy(
                sp_hbm.at[idx_v.at[row]],
                sbuf.at[pl.ds(j * 128, 128)], sems))
        for c in copies:
            c.wait()
        pltpu.sync_copy(xbuf, outx_hbm.at[pl.ds(base + r * CHUNK, CHUNK)])
        pltpu.sync_copy(sbuf, outs_hbm.at[pl.ds(base + r * CHUNK, CHUNK)])
        return 0

    lax.fori_loop(0, ROUNDS, round_body, 0)


def _tc_kernel(xg_ref, sg_ref, q_ref, kpt_ref, w_ref, out_ref):
    s = sg_ref[...]
    q = q_ref[...]
    rel = (s.reshape(BQ, H, 128) - q[:, None, :]).reshape(BQ * H, 128)
    rel2 = jnp.sum(rel * rel, axis=1, keepdims=True) - 1.0
    mm = jnp.dot(rel, kpt_ref[...], preferred_element_type=jnp.float32,
                 precision=lax.Precision.HIGHEST)
    sq_d = jnp.maximum(rel2 + mm, 0.0)
    w = jnp.maximum(1.0 - jnp.sqrt(sq_d) * (1.0 / KP_EXTENT), 0.0)
    w3 = w.reshape(BQ, H, 32)
    x3 = xg_ref[...].reshape(BQ, H, IN)
    g = lax.dot_general(w3, x3, (((1,), (1,)), ((0,), (0,))),
                        preferred_element_type=jnp.float32)
    acc = jnp.zeros((BQ, OUT), dtype=jnp.float32)
    for k in range(K):
        acc += jnp.dot(g[:, k, :], w_ref[k],
                       preferred_element_type=jnp.float32)
    out_ref[...] = acc


def kernel(q_pts, s_pts, neighb_inds, x, weights, kernel_points):
    idx = neighb_inds.astype(jnp.int32).reshape(-1)
    idx = jnp.concatenate(
        [idx, jnp.zeros((B - N * H,), dtype=jnp.int32)])
    idx = idx.reshape(NW, IDX_ROWS, 128)

    sp128 = jnp.concatenate(
        [s_pts, jnp.ones((M, 1), jnp.float32),
         jnp.zeros((M, 124), jnp.float32)], axis=1)
    q128 = jnp.concatenate(
        [q_pts, jnp.zeros((N, 125), jnp.float32)], axis=1)
    q128 = jnp.concatenate(
        [q128, jnp.zeros((NP_ - N, 128), jnp.float32)], axis=0)

    kp2 = jnp.sum(kernel_points * kernel_points, axis=1)
    kpt = jnp.zeros((128, 32), jnp.float32)
    kpt = kpt.at[:3, :K].set(-2.0 * kernel_points.T)
    kpt = kpt.at[3, :K].set(kp2)
    kpt = kpt.at[3, K:].set(1e9)

    xg, sg = pl.kernel(
        _sc_gather_kernel,
        out_type=(
            jax.ShapeDtypeStruct((B, IN), jnp.float32),
            jax.ShapeDtypeStruct((B, 128), jnp.float32),
        ),
        mesh=plsc.VectorSubcoreMesh(core_axis_name="c", subcore_axis_name="s"),
        scratch_types=[
            pltpu.VMEM((IDX_ROWS, 128), jnp.int32),
            pltpu.VMEM((CHUNK, IN), jnp.float32),
            pltpu.VMEM((CHUNK, 128), jnp.float32),
            pltpu.SemaphoreType.DMA,
            pltpu.SemaphoreType.DMA,
        ],
    )(x, sp128, idx)

    fx = pl.pallas_call(
        _tc_kernel,
        grid=(NBLK,),
        in_specs=[
            pl.BlockSpec((BQ * H, IN), lambda i: (i, 0)),
            pl.BlockSpec((BQ * H, 128), lambda i: (i, 0)),
            pl.BlockSpec((BQ, 128), lambda i: (i, 0)),
            pl.BlockSpec((128, 32), lambda i: (0, 0)),
            pl.BlockSpec((K, IN, OUT), lambda i: (0, 0, 0)),
        ],
        out_specs=pl.BlockSpec((BQ, OUT), lambda i: (i, 0)),
        out_shape=jax.ShapeDtypeStruct((NP_, OUT), jnp.float32),
    )(xg, sg, q128, kpt, weights)

    return fx[:N]


# speedup vs baseline: 1.9158x; 1.1823x over previous
import math

import jax
import jax.numpy as jnp
from jax import lax
from jax.experimental import pallas as pl
from jax.experimental.pallas import tpu as pltpu
from jax.experimental.pallas import tpu_sc as plsc

N = 10000
M = 10000
H = 32
IN = 128
OUT = 128
K = 27
KS = 3
P = 3
RADIUS = 0.5
KP_EXTENT = 2.0 * RADIUS / (KS - 1) / math.sqrt(P)

BQ = 256
NP_ = 10240
NBLK = NP_ // BQ
B = NP_ * H

NW = 32
NC = 2
BPW = B // NW
CHUNK = 256
ROUNDS = BPW // CHUNK
IDX_ROWS = BPW // 128
IPR = IDX_ROWS // ROUNDS


def _sc_gather_kernel(x_hbm, sx_hbm, sy_hbm, sz_hbm, idx_hbm,
                      outx_hbm, osx_hbm, osy_hbm, osz_hbm,
                      idxv, buf0, buf1, sgx, sgy, sgz,
                      gsem0, gsem1, wsem):
    wid = lax.axis_index("s") * NC + lax.axis_index("c")
    base = wid * BPW
    pltpu.sync_copy(idx_hbm.at[wid], idxv)
    bufs = (buf0, buf1)
    gsems = (gsem0, gsem1)
    cols = ((sx_hbm, sgx), (sy_hbm, sgy), (sz_hbm, sgz))

    def fire(r, b):
        for j in range(IPR):
            pltpu.async_copy(
                x_hbm.at[idxv.at[r * IPR + j]],
                bufs[b].at[pl.ds(j * 128, 128)], gsems[b])

    fire(0, 0)
    fire(1, 1)

    def round_pair(rr, _):
        for b in range(2):
            r = rr * 2 + b
            for j in range(IPR):
                row = r * IPR + j
                for col_hbm, sg in cols:
                    pltpu.async_copy(col_hbm.at[idxv.at[row]],
                                     sg.at[pl.ds(row * 128, 128)], wsem)
            for j in range(IPR):
                pltpu.make_async_copy(
                    x_hbm.at[idxv.at[r * IPR + j]],
                    bufs[b].at[pl.ds(j * 128, 128)], gsems[b]).wait()
            pltpu.sync_copy(bufs[b],
                            outx_hbm.at[pl.ds(base + r * CHUNK, CHUNK)])
            pl.when(r < ROUNDS - 2)(lambda: fire(r + 2, b))
        return 0

    lax.fori_loop(0, ROUNDS // 2, round_pair, 0)

    def drain(row, _):
        for col_hbm, sg in cols:
            pltpu.make_async_copy(col_hbm.at[idxv.at[row]],
                                  sg.at[pl.ds(row * 128, 128)], wsem).wait()
        return 0

    lax.fori_loop(0, IDX_ROWS, drain, 0)
    for (_, sg), out in zip(cols, (osx_hbm, osy_hbm, osz_hbm)):
        pltpu.sync_copy(sg, out.at[pl.ds(base, BPW)])


def _tc_kernel(xg_ref, sg_ref, q_ref, kpt_ref, w_ref, out_ref):
    s = sg_ref[...]
    q = q_ref[...]
    rel = (s.reshape(BQ, H, 8) - q[:, None, :]).reshape(BQ * H, 8)
    rel2 = jnp.sum(rel * rel, axis=1, keepdims=True) - 1.0
    mm = jnp.dot(rel, kpt_ref[...], preferred_element_type=jnp.float32,
                 precision=lax.Precision.HIGHEST)
    sq_d = jnp.maximum(rel2 + mm, 0.0)
    w = jnp.maximum(1.0 - jnp.sqrt(sq_d) * (1.0 / KP_EXTENT), 0.0)
    w3 = w.reshape(BQ, H, 32)
    x3 = xg_ref[...].reshape(BQ, H, IN)
    g = lax.dot_general(w3, x3, (((1,), (1,)), ((0,), (0,))),
                        preferred_element_type=jnp.float32)
    acc = jnp.zeros((BQ, OUT), dtype=jnp.float32)
    for k in range(K):
        acc += jnp.dot(g[:, k, :], w_ref[k],
                       preferred_element_type=jnp.float32)
    out_ref[...] = acc


def kernel(q_pts, s_pts, neighb_inds, x, weights, kernel_points):
    idx = neighb_inds.astype(jnp.int32).reshape(-1)
    idx = jnp.concatenate(
        [idx, jnp.zeros((B - N * H,), dtype=jnp.int32)])
    idx3 = idx.reshape(NW, IDX_ROWS, 128)

    sx, sy, sz = s_pts[:, 0], s_pts[:, 1], s_pts[:, 2]

    q8 = jnp.concatenate(
        [q_pts, jnp.zeros((N, 5), jnp.float32)], axis=1)
    q8 = jnp.concatenate(
        [q8, jnp.zeros((NP_ - N, 8), jnp.float32)], axis=0)

    kp2 = jnp.sum(kernel_points * kernel_points, axis=1)
    kpt = jnp.zeros((8, 32), jnp.float32)
    kpt = kpt.at[:3, :K].set(-2.0 * kernel_points.T)
    kpt = kpt.at[3, :K].set(kp2)
    kpt = kpt.at[3, K:].set(1e9)

    mesh = plsc.VectorSubcoreMesh(core_axis_name="c", subcore_axis_name="s")

    xg, sgx, sgy, sgz = pl.kernel(
        _sc_gather_kernel,
        out_type=(
            jax.ShapeDtypeStruct((B, IN), jnp.float32),
            jax.ShapeDtypeStruct((B,), jnp.float32),
            jax.ShapeDtypeStruct((B,), jnp.float32),
            jax.ShapeDtypeStruct((B,), jnp.float32),
        ),
        mesh=mesh,
        scratch_types=[
            pltpu.VMEM((IDX_ROWS, 128), jnp.int32),
            pltpu.VMEM((CHUNK, IN), jnp.float32),
            pltpu.VMEM((CHUNK, IN), jnp.float32),
            pltpu.VMEM((BPW,), jnp.float32),
            pltpu.VMEM((BPW,), jnp.float32),
            pltpu.VMEM((BPW,), jnp.float32),
            pltpu.SemaphoreType.DMA,
            pltpu.SemaphoreType.DMA,
            pltpu.SemaphoreType.DMA,
        ],
    )(x, sx, sy, sz, idx3)

    s8 = jnp.stack(
        [sgx, sgy, sgz, jnp.ones((B,), jnp.float32)] +
        [jnp.zeros((B,), jnp.float32)] * 4, axis=1)

    fx = pl.pallas_call(
        _tc_kernel,
        grid=(NBLK,),
        in_specs=[
            pl.BlockSpec((BQ * H, IN), lambda i: (i, 0)),
            pl.BlockSpec((BQ * H, 8), lambda i: (i, 0)),
            pl.BlockSpec((BQ, 8), lambda i: (i, 0)),
            pl.BlockSpec((8, 32), lambda i: (0, 0)),
            pl.BlockSpec((K, IN, OUT), lambda i: (0, 0, 0)),
        ],
        out_specs=pl.BlockSpec((BQ, OUT), lambda i: (i, 0)),
        out_shape=jax.ShapeDtypeStruct((NP_, OUT), jnp.float32),
    )(xg, s8, q8, kpt, weights)

    return fx[:N]
